# Initial kernel scaffold; baseline (speedup 1.0000x reference)
import jax
import jax.numpy as jnp
from jax.experimental import pallas as pl


def kernel(x, img_size, W1, b1, Ws, bs, Wl, bl):
    raise NotImplementedError("write your pallas kernel here")


# speedup vs baseline: 109.4757x; 109.4757x over previous
import functools

import jax
import jax.numpy as jnp
import numpy as np
from jax.experimental import pallas as pl
from jax.experimental.pallas import tpu as pltpu
from jax.experimental.pallas import tpu_sc as plsc

_THRESH = 0.7
_N_PRE = 12000
_N_POST = 2000
_MIN_SIZE = 16.0
_STRIDE = 16
_LANE = 128


def _anchor_np(hh, ww):
    base_size, ratios, scales = 16, (0.5, 1, 2), (8, 16, 32)
    py = base_size / 2.0
    px = base_size / 2.0
    ab = np.zeros((len(ratios) * len(scales), 4), dtype=np.float32)
    for i in range(len(ratios)):
        for j in range(len(scales)):
            h = base_size * scales[j] * np.sqrt(ratios[i])
            w = base_size * scales[j] * np.sqrt(1.0 / ratios[i])
            idx = i * len(scales) + j
            ab[idx, 0] = py - h / 2.0
            ab[idx, 1] = px - w / 2.0
            ab[idx, 2] = py + h / 2.0
            ab[idx, 3] = px + w / 2.0
    shift_y = np.arange(0, hh * _STRIDE, _STRIDE)
    shift_x = np.arange(0, ww * _STRIDE, _STRIDE)
    sx, sy = np.meshgrid(shift_x, shift_y)
    shift = np.stack((sy.ravel(), sx.ravel(), sy.ravel(), sx.ravel()), axis=1)
    A = ab.shape[0]
    K = shift.shape[0]
    anc = ab.reshape((1, A, 4)) + shift.reshape((1, K, 4)).transpose((1, 0, 2))
    return anc.reshape((K * A, 4)).astype(np.float32)


def _loc2bbox(src, loc):
    sh = src[:, 2] - src[:, 0]
    sw = src[:, 3] - src[:, 1]
    scy = src[:, 0] + 0.5 * sh
    scx = src[:, 1] + 0.5 * sw
    dy = loc[:, 0]
    dx = loc[:, 1]
    dh = loc[:, 2]
    dw = loc[:, 3]
    cy = dy * sh + scy
    cx = dx * sw + scx
    h = jnp.exp(dh) * sh
    w = jnp.exp(dw) * sw
    return jnp.stack([cy - 0.5 * h, cx - 0.5 * w, cy + 0.5 * h, cx + 0.5 * w], axis=1)


def _conv2d(x, W, b, pad):
    y = jax.lax.conv_general_dilated(x, W, (1, 1), pad, dimension_numbers=("NCHW", "OIHW", "NCHW"))
    return y + b[None, :, None, None]


def _make_nms_kernel(nb, n_post, thresh):
    def nms_kernel(y1_ref, x1_ref, y2_ref, x2_ref, v_ref, keep_ref, supp_ref):
        keep_ref[...] = jnp.zeros_like(keep_ref)
        supp_ref[...] = jnp.zeros_like(supp_ref)
        subi = jax.lax.broadcasted_iota(jnp.int32, (_LANE, _LANE), 0)
        lane = jax.lax.broadcasted_iota(jnp.int32, (_LANE, _LANE), 1)
        eye = (subi == lane).astype(jnp.float32)
        tril = (subi < lane).astype(jnp.float32)

        def tcol(row):
            return jnp.sum(eye * row, axis=1, keepdims=True)

        def iou_gt(cy1, cx1, cy2, cx2, ry1, rx1, ry2, rx2):
            yy1 = jnp.maximum(cy1, ry1)
            xx1 = jnp.maximum(cx1, rx1)
            yy2 = jnp.minimum(cy2, ry2)
            xx2 = jnp.minimum(cx2, rx2)
            inter = jnp.maximum(yy2 - yy1, 0.0) * jnp.maximum(xx2 - xx1, 0.0)
            a_c = (cy2 - cy1) * (cx2 - cx1)
            a_r = (ry2 - ry1) * (rx2 - rx1)
            iou = inter / jnp.maximum(a_c + a_r - inter, 1e-9)
            return (iou > thresh).astype(jnp.float32)

        def outer_cond(c):
            k, count = c
            return jnp.logical_and(k < nb, count < n_post)

        def outer_body(c):
            k, count = c
            ry1 = y1_ref[pl.ds(k, 1), :]
            rx1 = x1_ref[pl.ds(k, 1), :]
            ry2 = y2_ref[pl.ds(k, 1), :]
            rx2 = x2_ref[pl.ds(k, 1), :]
            rv = v_ref[pl.ds(k, 1), :]
            base_row = rv * (1.0 - supp_ref[pl.ds(k, 1), :])
            cy1 = tcol(ry1)
            cx1 = tcol(rx1)
            cy2 = tcol(ry2)
            cx2 = tcol(rx2)
            Gl = iou_gt(cy1, cx1, cy2, cx2, ry1, rx1, ry2, rx2) * tril

            def fx_cond(fc):
                _, changed = fc
                return changed

            def fx_body(fc):
                keep_row, _ = fc
                sup = jnp.max(Gl * tcol(keep_row), axis=0, keepdims=True)
                new = base_row * (1.0 - sup)
                return new, jnp.any(new != keep_row)

            keep_row, _ = jax.lax.while_loop(fx_cond, fx_body, (base_row, jnp.bool_(True)))
            keep_ref[pl.ds(k, 1), :] = keep_row
            kc = tcol(keep_row)

            def fwd_body(l, _):
                ly1 = y1_ref[pl.ds(l, 1), :]
                lx1 = x1_ref[pl.ds(l, 1), :]
                ly2 = y2_ref[pl.ds(l, 1), :]
                lx2 = x2_ref[pl.ds(l, 1), :]
                g = iou_gt(cy1, cx1, cy2, cx2, ly1, lx1, ly2, lx2)
                sup_l = jnp.max(g * kc, axis=0, keepdims=True)
                supp_ref[pl.ds(l, 1), :] = jnp.maximum(supp_ref[pl.ds(l, 1), :], sup_l)
                return 0

            jax.lax.fori_loop(k + 1, nb, fwd_body, 0)
            return k + 1, count + jnp.sum(keep_row).astype(jnp.int32)

        jax.lax.while_loop(outer_cond, outer_body, (jnp.int32(0), jnp.int32(0)))

    return nms_kernel


def _nms_pallas(y1s, x1s, y2s, x2s, vs, nb, n_post, thresh, interpret=False):
    n, rows, _ = y1s.shape
    spec = pl.BlockSpec((None, rows, _LANE), lambda i: (i, 0, 0))
    return pl.pallas_call(
        _make_nms_kernel(nb, n_post, thresh),
        grid=(n,),
        in_specs=[spec] * 5,
        out_specs=spec,
        out_shape=jax.ShapeDtypeStruct((n, rows, _LANE), jnp.float32),
        scratch_shapes=[pltpu.VMEM((rows, _LANE), jnp.float32)],
        interpret=interpret,
    )(y1s, x1s, y2s, x2s, vs)


def _sc_gather_rows(table, idx):
    info = plsc.get_sparse_core_info()
    nc, ns = info.num_cores, info.num_subcores
    nw = nc * ns
    B = idx.shape[0]
    b_per_w = B // nw
    mesh = plsc.VectorSubcoreMesh(core_axis_name="c", subcore_axis_name="s")

    @functools.partial(
        pl.kernel,
        mesh=mesh,
        out_type=jax.ShapeDtypeStruct((B, table.shape[1]), jnp.float32),
        scratch_types=[
            pltpu.VMEM((b_per_w,), jnp.int32),
            pltpu.VMEM((b_per_w, table.shape[1]), jnp.float32),
            pltpu.SemaphoreType.DMA,
        ],
    )
    def k(table_hbm, idx_hbm, out_hbm, idx_v, rows_v, sem):
        wid = jax.lax.axis_index("s") * nc + jax.lax.axis_index("c")
        base = wid * b_per_w
        pltpu.sync_copy(idx_hbm.at[pl.ds(base, b_per_w)], idx_v)
        pltpu.async_copy(table_hbm.at[idx_v], rows_v, sem).wait()
        pltpu.sync_copy(rows_v, out_hbm.at[pl.ds(base, b_per_w)])

    return k(table, idx)


def kernel(x, img_size, W1, b1, Ws, bs, Wl, bl):
    n, _, hh, ww = x.shape
    anchor = jnp.asarray(_anchor_np(hh, ww))
    n_anchor = 9
    h = jax.nn.relu(_conv2d(x, W1, b1, "SAME"))
    rpn_locs = _conv2d(h, Wl, bl, "VALID").transpose(0, 2, 3, 1).reshape(n, -1, 4)
    rpn_scores_map = _conv2d(h, Ws, bs, "VALID").transpose(0, 2, 3, 1)
    soft = jax.nn.softmax(rpn_scores_map.reshape(n, hh, ww, n_anchor, 2), axis=4)
    rpn_fg = soft[..., 1].reshape(n, -1)
    rpn_scores = rpn_scores_map.reshape(n, -1, 2)
    H = img_size[0].astype(jnp.float32)
    W = img_size[1].astype(jnp.float32)

    nb = -(-_N_PRE // _LANE)
    rows = -(-nb // 8) * 8
    pad = rows * _LANE - _N_PRE

    n_total = anchor.shape[0]
    v_pad = -(-n_total // 8) * 8 - n_total
    b_sorted = -(-_N_PRE // 256) * 256

    coords, valids, roi_s_list = [], [], []
    for i in range(n):
        roi = _loc2bbox(anchor, rpn_locs[i])
        y1 = jnp.clip(roi[:, 0], 0.0, H)
        x1 = jnp.clip(roi[:, 1], 0.0, W)
        y2 = jnp.clip(roi[:, 2], 0.0, H)
        x2 = jnp.clip(roi[:, 3], 0.0, W)
        roi = jnp.stack([y1, x1, y2, x2], axis=1)
        hs = roi[:, 2] - roi[:, 0]
        ws_ = roi[:, 3] - roi[:, 1]
        valid = (hs >= _MIN_SIZE) & (ws_ >= _MIN_SIZE)
        score = jnp.where(valid, rpn_fg[i], -jnp.inf)
        order = jnp.argsort(-score)[:_N_PRE]
        table = jnp.concatenate(
            [roi, valid.astype(jnp.float32)[:, None], jnp.zeros((n_total, 123), jnp.float32)],
            axis=1,
        )
        table = jnp.pad(table, ((0, v_pad), (0, 0)))
        idx_p = jnp.pad(order.astype(jnp.int32), (0, b_sorted - _N_PRE))
        g = _sc_gather_rows(table, idx_p)
        real = jnp.arange(b_sorted) < _N_PRE
        roi_s = g[:_N_PRE, :4]
        valid_s = jnp.where(real, g[:, 4] > 0.5, False)
        roi_s_list.append(roi_s)
        cpad = jnp.pad(g[:, :4], ((0, rows * _LANE - b_sorted), (0, 0))).reshape(rows, _LANE, 4)
        coords.append(cpad)
        valids.append(
            jnp.pad(valid_s.astype(jnp.float32), (0, rows * _LANE - b_sorted)).reshape(rows, _LANE)
        )

    cs = jnp.stack(coords)
    vs = jnp.stack(valids)
    keep = _nms_pallas(
        cs[..., 0], cs[..., 1], cs[..., 2], cs[..., 3], vs, nb, _N_POST, _THRESH
    )

    rois, roi_indices = [], []
    for i in range(n):
        keep_i = keep[i].reshape(-1)[:_N_PRE] > 0.5
        idx = jnp.nonzero(keep_i, size=_N_POST, fill_value=0)[0]
        rois.append(roi_s_list[i][idx])
        roi_indices.append(jnp.full((_N_POST,), i, dtype=jnp.int32))
    rois = jnp.concatenate(rois, axis=0)
    roi_indices = jnp.concatenate(roi_indices, axis=0)
    return (rpn_locs, rpn_scores, rois, roi_indices, anchor)
